# Initial kernel scaffold; baseline (speedup 1.0000x reference)
import jax
import jax.numpy as jnp
from jax.experimental import pallas as pl


def kernel(input_ids, word_emb, pos_emb, type_emb, ln_gamma, ln_beta):
    raise NotImplementedError("write your pallas kernel here")


# speedup vs baseline: 2.7864x; 2.7864x over previous
import functools

import jax
import jax.numpy as jnp
from jax import lax
from jax.experimental import pallas as pl
from jax.experimental.pallas import tpu as pltpu
from jax.experimental.pallas import tpu_sc as plsc

VOCAB = 1000000
HIDDEN = 128
B = 1024
S = 200

NC = 2
NS = 16
NW = NC * NS
L = 16
NV = HIDDEN // L

CHUNK = 100
N_TOK = B * S
N_CHUNKS = N_TOK // CHUNK
CHUNKS_PER_W = N_CHUNKS // NW


_GATHER_DNUMS = lax.GatherDimensionNumbers(
    offset_dims=(), collapsed_slice_dims=(0,), start_index_map=(0,))


def _shuffle(v, d):
    idx = (lax.iota(jnp.int32, L) ^ jnp.int32(d))[:, None]
    return lax.gather(v, idx, _GATHER_DNUMS, slice_sizes=(1,),
                      mode=lax.GatherScatterMode.PROMISE_IN_BOUNDS)


def _splat_sum(v):
    for d in (1, 2, 4, 8):
        v = v + _shuffle(v, d)
    return v


def _rsqrt(v):
    i = lax.bitcast_convert_type(v, jnp.int32)
    i = jnp.int32(0x5F3759DF) - (i >> 1)
    r = lax.bitcast_convert_type(i, jnp.float32)
    half_v = 0.5 * v
    for _ in range(3):
        r = r * (1.5 - half_v * r * r)
    return r


def _sc_kernel(ids_hbm, word_hbm, pos_hbm, type_hbm, gamma_hbm, beta_hbm,
               out_hbm, comb_v, rows_v, idx_v, gb_v, trow_v, sem):
    wid = lax.axis_index("s") * NC + lax.axis_index("c")

    pltpu.sync_copy(pos_hbm.at[pl.ds(0, S)], comb_v)
    pltpu.sync_copy(type_hbm.at[pl.ds(0, 1)], trow_v)
    pltpu.sync_copy(gamma_hbm, gb_v.at[0])
    pltpu.sync_copy(beta_hbm, gb_v.at[1])

    trow = [trow_v[0, pl.ds(L * j, L)] for j in range(NV)]

    def fold_type(s, carry):
        for j in range(NV):
            comb_v[s, pl.ds(L * j, L)] = comb_v[s, pl.ds(L * j, L)] + trow[j]
        return carry

    lax.fori_loop(0, S, fold_type, 0)

    gam = [gb_v[0, pl.ds(L * j, L)] for j in range(NV)]
    bet = [gb_v[1, pl.ds(L * j, L)] for j in range(NV)]

    inv_h = jnp.float32(1.0 / HIDDEN)

    def do_chunk(q, carry):
        cid = wid * CHUNKS_PER_W + q
        sbase = (q % 2) * CHUNK

        pltpu.sync_copy(ids_hbm.at[cid], idx_v)
        pltpu.async_copy(word_hbm.at[idx_v], rows_v, sem).wait()

        def token(t, c2):
            x = [rows_v[t, pl.ds(L * j, L)] + comb_v[sbase + t, pl.ds(L * j, L)]
                 for j in range(NV)]
            s01 = x[0] + x[1]
            s23 = x[2] + x[3]
            s45 = x[4] + x[5]
            s67 = x[6] + x[7]
            svec = (s01 + s23) + (s45 + s67)
            m = [xj * xj for xj in x]
            m01 = m[0] + m[1]
            m23 = m[2] + m[3]
            m45 = m[4] + m[5]
            m67 = m[6] + m[7]
            mvec = (m01 + m23) + (m45 + m67)
            total = _splat_sum(svec)
            ssq = _splat_sum(mvec)
            mean = total * inv_h
            var = ssq * inv_h - mean * mean
            r = _rsqrt(var + jnp.float32(1e-12))
            for j in range(NV):
                rows_v[t, pl.ds(L * j, L)] = (x[j] - mean) * (gam[j] * r) + bet[j]
            return c2

        lax.fori_loop(0, CHUNK, token, 0)
        pltpu.sync_copy(rows_v, out_hbm.at[cid])
        return carry

    lax.fori_loop(0, CHUNKS_PER_W, do_chunk, 0)


@jax.jit
def _run(ids2d, word_emb, pos_emb, type_emb, ln_gamma, ln_beta):
    mesh = plsc.VectorSubcoreMesh(core_axis_name="c", subcore_axis_name="s")
    f = pl.kernel(
        _sc_kernel,
        out_type=jax.ShapeDtypeStruct((N_CHUNKS, CHUNK, HIDDEN), jnp.float32),
        mesh=mesh,
        scratch_types=[
            pltpu.VMEM((S, HIDDEN), jnp.float32),
            pltpu.VMEM((CHUNK, HIDDEN), jnp.float32),
            pltpu.VMEM((CHUNK,), jnp.int32),
            pltpu.VMEM((2, HIDDEN), jnp.float32),
            pltpu.VMEM((1, HIDDEN), jnp.float32),
            pltpu.SemaphoreType.DMA,
        ],
    )
    return f(ids2d, word_emb, pos_emb, type_emb, ln_gamma, ln_beta)


def kernel(input_ids, word_emb, pos_emb, type_emb, ln_gamma, ln_beta):
    ids2d = input_ids.astype(jnp.int32).reshape(N_CHUNKS, CHUNK)
    out = _run(ids2d, word_emb, pos_emb, type_emb, ln_gamma, ln_beta)
    return out.reshape(B, S, HIDDEN)


# speedup vs baseline: 5.4339x; 1.9501x over previous
import jax
import jax.numpy as jnp
from jax import lax
from jax.experimental import pallas as pl
from jax.experimental.pallas import tpu as pltpu
from jax.experimental.pallas import tpu_sc as plsc

VOCAB = 1000000
HIDDEN = 128
B = 1024
S = 200

NC = 2
NS = 16
NW = NC * NS
L = 16
NV = HIDDEN // L

CHUNK = 100
N_TOK = B * S
N_CHUNKS = N_TOK // CHUNK
CHUNKS_PER_W = N_CHUNKS // NW
UNROLL = 2

_GATHER_DNUMS = lax.GatherDimensionNumbers(
    offset_dims=(), collapsed_slice_dims=(0,), start_index_map=(0,))


def _shuffle(v, d):
    idx = (lax.iota(jnp.int32, L) ^ jnp.int32(d))[:, None]
    return lax.gather(v, idx, _GATHER_DNUMS, slice_sizes=(1,),
                      mode=lax.GatherScatterMode.PROMISE_IN_BOUNDS)


def _splat_sum(v):
    for d in (1, 2, 4, 8):
        v = v + _shuffle(v, d)
    return v


def _rsqrt(v):
    i = lax.bitcast_convert_type(v, jnp.int32)
    i = jnp.int32(0x5F3759DF) - (i >> 1)
    r = lax.bitcast_convert_type(i, jnp.float32)
    half_v = 0.5 * v
    for _ in range(3):
        r = r * (1.5 - half_v * r * r)
    return r


def _sc_kernel(ids_hbm, word_hbm, pos_hbm, type_hbm, gamma_hbm, beta_hbm,
               out_hbm, comb_v, rows_v, obuf_v, idx_v, gb_v, trow_v,
               gsem0, gsem1, osem0, osem1):
    wid = lax.axis_index("s") * NC + lax.axis_index("c")
    cbase = wid * CHUNKS_PER_W
    gsem = (gsem0, gsem1)
    osem = (osem0, osem1)

    pltpu.sync_copy(pos_hbm.at[pl.ds(0, S)], comb_v)
    pltpu.sync_copy(type_hbm.at[pl.ds(0, 1)], trow_v)
    pltpu.sync_copy(gamma_hbm, gb_v.at[0])
    pltpu.sync_copy(beta_hbm, gb_v.at[1])

    trow = [trow_v[0, pl.ds(L * j, L)] for j in range(NV)]

    def fold_type(s, carry):
        for j in range(NV):
            comb_v[s, pl.ds(L * j, L)] = comb_v[s, pl.ds(L * j, L)] + trow[j]
        return carry

    lax.fori_loop(0, S, fold_type, 0)

    gam = [gb_v[0, pl.ds(L * j, L)] for j in range(NV)]
    bet = [gb_v[1, pl.ds(L * j, L)] for j in range(NV)]
    inv_h = jnp.float32(1.0 / HIDDEN)

    def fetch(q, buf):
        pltpu.sync_copy(ids_hbm.at[cbase + q], idx_v.at[buf])
        pltpu.async_copy(word_hbm.at[idx_v.at[buf]], rows_v.at[buf],
                         gsem[buf])

    fetch(0, 0)
    fetch(1, 1)

    def do_pair(p, carry):
        for buf in range(2):
            q = 2 * p + buf
            cid = cbase + q
            pltpu.make_async_copy(word_hbm.at[idx_v.at[buf]],
                                  rows_v.at[buf], gsem[buf]).wait()

            @pl.when(p > 0)
            def _():
                pltpu.make_async_copy(obuf_v.at[buf], out_hbm.at[cid - 2],
                                      osem[buf]).wait()

            def token_blk(i, c2):
                for k in range(UNROLL):
                    t = UNROLL * i + k
                    x = [rows_v[buf, t, pl.ds(L * j, L)]
                         + comb_v[buf * CHUNK + t, pl.ds(L * j, L)]
                         for j in range(NV)]
                    s01 = x[0] + x[1]
                    s23 = x[2] + x[3]
                    s45 = x[4] + x[5]
                    s67 = x[6] + x[7]
                    svec = (s01 + s23) + (s45 + s67)
                    m = [xj * xj for xj in x]
                    m01 = m[0] + m[1]
                    m23 = m[2] + m[3]
                    m45 = m[4] + m[5]
                    m67 = m[6] + m[7]
                    mvec = (m01 + m23) + (m45 + m67)
                    total = _splat_sum(svec)
                    ssq = _splat_sum(mvec)
                    mean = total * inv_h
                    var = ssq * inv_h - mean * mean
                    r = _rsqrt(var + jnp.float32(1e-12))
                    for j in range(NV):
                        obuf_v[buf, t, pl.ds(L * j, L)] = (
                            (x[j] - mean) * (gam[j] * r) + bet[j])
                return c2

            lax.fori_loop(0, CHUNK // UNROLL, token_blk, 0)

            pltpu.async_copy(obuf_v.at[buf], out_hbm.at[cid], osem[buf])

            @pl.when(p < CHUNKS_PER_W // 2 - 1)
            def _():
                fetch(q + 2, buf)
        return carry

    lax.fori_loop(0, CHUNKS_PER_W // 2, do_pair, 0)

    for buf in range(2):
        pltpu.make_async_copy(obuf_v.at[buf],
                              out_hbm.at[cbase + CHUNKS_PER_W - 2 + buf],
                              osem[buf]).wait()


@jax.jit
def _run(ids2d, word_emb, pos_emb, type_emb, ln_gamma, ln_beta):
    mesh = plsc.VectorSubcoreMesh(core_axis_name="c", subcore_axis_name="s")
    f = pl.kernel(
        _sc_kernel,
        out_type=jax.ShapeDtypeStruct((N_CHUNKS, CHUNK, HIDDEN), jnp.float32),
        mesh=mesh,
        scratch_types=[
            pltpu.VMEM((S, HIDDEN), jnp.float32),
            pltpu.VMEM((2, CHUNK, HIDDEN), jnp.float32),
            pltpu.VMEM((2, CHUNK, HIDDEN), jnp.float32),
            pltpu.VMEM((2, CHUNK), jnp.int32),
            pltpu.VMEM((2, HIDDEN), jnp.float32),
            pltpu.VMEM((1, HIDDEN), jnp.float32),
            pltpu.SemaphoreType.DMA,
            pltpu.SemaphoreType.DMA,
            pltpu.SemaphoreType.DMA,
            pltpu.SemaphoreType.DMA,
        ],
    )
    return f(ids2d, word_emb, pos_emb, type_emb, ln_gamma, ln_beta)


def kernel(input_ids, word_emb, pos_emb, type_emb, ln_gamma, ln_beta):
    ids2d = input_ids.astype(jnp.int32).reshape(N_CHUNKS, CHUNK)
    out = _run(ids2d, word_emb, pos_emb, type_emb, ln_gamma, ln_beta)
    return out.reshape(B, S, HIDDEN)
